# Initial kernel scaffold; baseline (speedup 1.0000x reference)
import functools

import jax
import jax.numpy as jnp
from jax import lax
from jax.experimental import pallas as pl
from jax.experimental.pallas import tpu as pltpu
from jax.experimental.pallas import tpu_sc as plsc

N = 10000
E = 320000
D = 128
NCLS = 40

N_PAD = 10240
D2 = 64
NW = 32
CHUNK = 128
CHUNKS = 80
E_PAD = NW * CHUNKS * CHUNK
RPT = N_PAD // 16
BLK = 1024


def _mesh():
    return plsc.VectorSubcoreMesh(core_axis_name="c", subcore_axis_name="s")


def _make_agg(d):
    @functools.partial(
        pl.kernel,
        out_type=jax.ShapeDtypeStruct((2, N_PAD, d), jnp.float32),
        mesh=_mesh(),
        scratch_types=[
            pltpu.VMEM((CHUNKS, CHUNK), jnp.int32),
            pltpu.VMEM((CHUNKS, CHUNK), jnp.int32),
            pltpu.VMEM((CHUNK, d), jnp.float32),
            pltpu.VMEM_SHARED((N_PAD, d), jnp.float32),
            pltpu.SemaphoreType.DMA,
        ],
    )
    def agg(y_hbm, src_hbm, dst_hbm, zeros_hbm, out_hbm,
            src_v, dst_v, rows_v, acc, sem):
        cid = lax.axis_index("c")
        sid = lax.axis_index("s")
        wid = sid * 2 + cid
        pltpu.sync_copy(src_hbm.at[pl.ds(wid * CHUNKS, CHUNKS)], src_v)
        pltpu.sync_copy(dst_hbm.at[pl.ds(wid * CHUNKS, CHUNKS)], dst_v)
        pltpu.sync_copy(zeros_hbm, acc.at[pl.ds(sid * RPT, RPT)])
        plsc.subcore_barrier()

        def body(j, carry):
            pltpu.async_copy(y_hbm.at[src_v.at[j]], rows_v, sem).wait()
            pltpu.sync_copy(rows_v, acc.at[dst_v.at[j]], add=True)
            return carry

        lax.fori_loop(0, CHUNKS, body, 0)
        plsc.subcore_barrier()
        pltpu.sync_copy(acc.at[pl.ds(sid * RPT, RPT)],
                        out_hbm.at[cid].at[pl.ds(sid * RPT, RPT)])

    return agg


_agg128 = _make_agg(D)
_agg64 = _make_agg(D2)


@functools.partial(
    pl.kernel,
    out_type=jax.ShapeDtypeStruct((2, 2, N_PAD, 16), jnp.float32),
    mesh=_mesh(),
    scratch_types=[
        pltpu.VMEM((CHUNKS, CHUNK), jnp.int32),
        pltpu.VMEM((CHUNKS, CHUNK), jnp.int32),
        pltpu.VMEM((CHUNK, 16), jnp.float32),
        pltpu.VMEM_SHARED((N_PAD, 16), jnp.float32),
        pltpu.VMEM_SHARED((N_PAD, 16), jnp.float32),
    ],
)
def _degrees(src_hbm, dst_hbm, ones_hbm, zeros_hbm, out_hbm,
             src_v, dst_v, ones_v, dego, degi):
    cid = lax.axis_index("c")
    sid = lax.axis_index("s")
    wid = sid * 2 + cid
    pltpu.sync_copy(src_hbm.at[pl.ds(wid * CHUNKS, CHUNKS)], src_v)
    pltpu.sync_copy(dst_hbm.at[pl.ds(wid * CHUNKS, CHUNKS)], dst_v)
    pltpu.sync_copy(ones_hbm, ones_v)
    pltpu.sync_copy(zeros_hbm, dego.at[pl.ds(sid * RPT, RPT)])
    pltpu.sync_copy(zeros_hbm, degi.at[pl.ds(sid * RPT, RPT)])
    plsc.subcore_barrier()

    def body(j, carry):
        pltpu.sync_copy(ones_v, dego.at[src_v.at[j]], add=True)
        pltpu.sync_copy(ones_v, degi.at[dst_v.at[j]], add=True)
        return carry

    lax.fori_loop(0, CHUNKS, body, 0)
    plsc.subcore_barrier()
    pltpu.sync_copy(dego.at[pl.ds(sid * RPT, RPT)],
                    out_hbm.at[cid].at[0].at[pl.ds(sid * RPT, RPT)])
    pltpu.sync_copy(degi.at[pl.ds(sid * RPT, RPT)],
                    out_hbm.at[cid].at[1].at[pl.ds(sid * RPT, RPT)])


def _norm(a_ref, b_ref):
    deg = a_ref[:, 0:1] + b_ref[:, 0:1]
    return lax.rsqrt(jnp.maximum(deg, 1.0))


def _first_pre_body(x_ref, do0, do1, w_ref, o_ref):
    ns = _norm(do0, do1)
    o_ref[...] = jnp.dot(x_ref[...] * ns, w_ref[...],
                         preferred_element_type=jnp.float32)


def _mid_pre_body(p0, p1, do0, do1, di0, di1, b_ref, w_ref, o_ref):
    nd = _norm(di0, di1)
    h = jnp.maximum((p0[...] + p1[...]) * nd + b_ref[...], 0.0)
    ns = _norm(do0, do1)
    o_ref[...] = jnp.dot(h * ns, w_ref[...],
                         preferred_element_type=jnp.float32)


def _final_body(p0, p1, di0, di1, b_ref, o_ref):
    nd = _norm(di0, di1)
    o_ref[...] = (p0[...] + p1[...]) * nd + b_ref[...]


def _row_spec(w):
    return pl.BlockSpec((BLK, w), lambda i: (i, 0))


def _full_spec(h, w):
    return pl.BlockSpec((h, w), lambda i: (0, 0))


_GRID = N_PAD // BLK


def _first_pre(x, do0, do1, w):
    return pl.pallas_call(
        _first_pre_body,
        grid=(_GRID,),
        in_specs=[_row_spec(D), _row_spec(16), _row_spec(16),
                  _full_spec(D, D)],
        out_specs=_row_spec(D),
        out_shape=jax.ShapeDtypeStruct((N_PAD, D), jnp.float32),
    )(x, do0, do1, w)


def _mid_pre(p0, p1, do0, do1, di0, di1, b, w, dout):
    return pl.pallas_call(
        _mid_pre_body,
        grid=(_GRID,),
        in_specs=[_row_spec(D), _row_spec(D), _row_spec(16), _row_spec(16),
                  _row_spec(16), _row_spec(16), _full_spec(1, D),
                  _full_spec(D, dout)],
        out_specs=_row_spec(dout),
        out_shape=jax.ShapeDtypeStruct((N_PAD, dout), jnp.float32),
    )(p0, p1, do0, do1, di0, di1, b, w)


def _final(p0, p1, di0, di1, b):
    return pl.pallas_call(
        _final_body,
        grid=(_GRID,),
        in_specs=[_row_spec(D2), _row_spec(D2), _row_spec(16), _row_spec(16),
                  _full_spec(1, D2)],
        out_specs=_row_spec(D2),
        out_shape=jax.ShapeDtypeStruct((N_PAD, D2), jnp.float32),
    )(p0, p1, di0, di1, b)


def kernel(features, edge_index, W0, b0, W1, b1, W2, b2):
    pad = jnp.full((E_PAD - E,), N, dtype=jnp.int32)
    src = jnp.concatenate([edge_index[0], pad]).reshape(E_PAD // CHUNK, CHUNK)
    dst = jnp.concatenate([edge_index[1], pad]).reshape(E_PAD // CHUNK, CHUNK)

    x = jnp.pad(features, ((0, N_PAD - N), (0, 0)))
    w2 = jnp.pad(W2, ((0, 0), (0, D2 - NCLS)))
    b0r = b0.reshape(1, D)
    b1r = b1.reshape(1, D)
    b2r = jnp.pad(b2, (0, D2 - NCLS)).reshape(1, D2)

    ones16 = jnp.ones((CHUNK, 16), jnp.float32)
    z16 = jnp.zeros((RPT, 16), jnp.float32)
    z128 = jnp.zeros((RPT, D), jnp.float32)
    z64 = jnp.zeros((RPT, D2), jnp.float32)

    deg = _degrees(src, dst, ones16, z16)
    do0, do1 = deg[0, 0], deg[1, 0]
    di0, di1 = deg[0, 1], deg[1, 1]

    y0 = _first_pre(x, do0, do1, W0)
    p = _agg128(y0, src, dst, z128)
    y1 = _mid_pre(p[0], p[1], do0, do1, di0, di1, b0r, W1, D)
    p = _agg128(y1, src, dst, z128)
    y2 = _mid_pre(p[0], p[1], do0, do1, di0, di1, b1r, w2, D2)
    p = _agg64(y2, src, dst, z64)
    out = _final(p[0], p[1], di0, di1, b2r)
    return out[:N, :NCLS]


# speedup vs baseline: 3.6115x; 3.6115x over previous
import functools

import jax
import jax.numpy as jnp
from jax import lax
from jax.experimental import pallas as pl
from jax.experimental.pallas import tpu as pltpu
from jax.experimental.pallas import tpu_sc as plsc

N = 10000
E = 320000
D = 128
NCLS = 40

N_PAD = 10240
D2 = 64
NW = 32
CHUNK = 128
CHUNKS = 80
E_PAD = NW * CHUNKS * CHUNK
RPT = N_PAD // 16
BLK = 1024


def _mesh():
    return plsc.VectorSubcoreMesh(core_axis_name="c", subcore_axis_name="s",
                                  num_cores=2, num_subcores=16)


def _make_agg(d):
    @functools.partial(
        pl.kernel,
        out_type=jax.ShapeDtypeStruct((2, N_PAD, d), jnp.float32),
        mesh=_mesh(),
        compiler_params=pltpu.CompilerParams(
            use_tc_tiling_on_sc=(d % 128 == 0)),
        scratch_types=[
            pltpu.VMEM((CHUNKS, CHUNK), jnp.int32),
            pltpu.VMEM((CHUNKS, CHUNK), jnp.int32),
            pltpu.VMEM((CHUNK, d), jnp.float32),
            pltpu.VMEM_SHARED((N_PAD, d), jnp.float32),
            pltpu.SemaphoreType.DMA,
        ],
    )
    def agg(y_hbm, src_hbm, dst_hbm, zeros_hbm, out_hbm,
            src_v, dst_v, rows_v, acc, sem):
        cid = lax.axis_index("c")
        sid = lax.axis_index("s")
        wid = sid * 2 + cid
        pltpu.sync_copy(src_hbm.at[pl.ds(wid * CHUNKS, CHUNKS)], src_v)
        pltpu.sync_copy(dst_hbm.at[pl.ds(wid * CHUNKS, CHUNKS)], dst_v)
        pltpu.sync_copy(zeros_hbm, acc.at[pl.ds(sid * RPT, RPT)])
        plsc.subcore_barrier()

        def body(j, carry):
            pltpu.async_copy(y_hbm.at[src_v.at[j]], rows_v, sem).wait()
            pltpu.sync_copy(rows_v, acc.at[dst_v.at[j]], add=True)
            return carry

        lax.fori_loop(0, CHUNKS, body, 0)
        plsc.subcore_barrier()
        pltpu.sync_copy(acc.at[pl.ds(sid * RPT, RPT)],
                        out_hbm.at[cid].at[pl.ds(sid * RPT, RPT)])

    return agg


_agg128 = _make_agg(D)
_agg64 = _make_agg(D2)


@functools.partial(
    pl.kernel,
    out_type=jax.ShapeDtypeStruct((2, 2, N_PAD, 16), jnp.float32),
    mesh=_mesh(),
    compiler_params=pltpu.CompilerParams(use_tc_tiling_on_sc=False),
    scratch_types=[
        pltpu.VMEM((CHUNKS, CHUNK), jnp.int32),
        pltpu.VMEM((CHUNKS, CHUNK), jnp.int32),
        pltpu.VMEM((CHUNK, 16), jnp.float32),
        pltpu.VMEM_SHARED((N_PAD, 16), jnp.float32),
        pltpu.VMEM_SHARED((N_PAD, 16), jnp.float32),
    ],
)
def _degrees(src_hbm, dst_hbm, ones_hbm, zeros_hbm, out_hbm,
             src_v, dst_v, ones_v, dego, degi):
    cid = lax.axis_index("c")
    sid = lax.axis_index("s")
    wid = sid * 2 + cid
    pltpu.sync_copy(src_hbm.at[pl.ds(wid * CHUNKS, CHUNKS)], src_v)
    pltpu.sync_copy(dst_hbm.at[pl.ds(wid * CHUNKS, CHUNKS)], dst_v)
    pltpu.sync_copy(ones_hbm, ones_v)
    pltpu.sync_copy(zeros_hbm, dego.at[pl.ds(sid * RPT, RPT)])
    pltpu.sync_copy(zeros_hbm, degi.at[pl.ds(sid * RPT, RPT)])
    plsc.subcore_barrier()

    def body(j, carry):
        pltpu.sync_copy(ones_v, dego.at[src_v.at[j]], add=True)
        pltpu.sync_copy(ones_v, degi.at[dst_v.at[j]], add=True)
        return carry

    lax.fori_loop(0, CHUNKS, body, 0)
    plsc.subcore_barrier()
    pltpu.sync_copy(dego.at[pl.ds(sid * RPT, RPT)],
                    out_hbm.at[cid].at[0].at[pl.ds(sid * RPT, RPT)])
    pltpu.sync_copy(degi.at[pl.ds(sid * RPT, RPT)],
                    out_hbm.at[cid].at[1].at[pl.ds(sid * RPT, RPT)])


def _norm(a_ref, b_ref):
    deg = a_ref[:, 0:1] + b_ref[:, 0:1]
    return lax.rsqrt(jnp.maximum(deg, 1.0))


def _first_pre_body(x_ref, do0, do1, w_ref, o_ref):
    ns = _norm(do0, do1)
    o_ref[...] = jnp.dot(x_ref[...] * ns, w_ref[...],
                         preferred_element_type=jnp.float32)


def _mid_pre_body(p0, p1, do0, do1, di0, di1, b_ref, w_ref, o_ref):
    nd = _norm(di0, di1)
    h = jnp.maximum((p0[...] + p1[...]) * nd + b_ref[...], 0.0)
    ns = _norm(do0, do1)
    o_ref[...] = jnp.dot(h * ns, w_ref[...],
                         preferred_element_type=jnp.float32)


def _final_body(p0, p1, di0, di1, b_ref, o_ref):
    nd = _norm(di0, di1)
    o_ref[...] = (p0[...] + p1[...]) * nd + b_ref[...]


def _row_spec(w):
    return pl.BlockSpec((BLK, w), lambda i: (i, 0))


def _full_spec(h, w):
    return pl.BlockSpec((h, w), lambda i: (0, 0))


_GRID = N_PAD // BLK


def _first_pre(x, do0, do1, w):
    return pl.pallas_call(
        _first_pre_body,
        grid=(_GRID,),
        in_specs=[_row_spec(D), _row_spec(16), _row_spec(16),
                  _full_spec(D, D)],
        out_specs=_row_spec(D),
        out_shape=jax.ShapeDtypeStruct((N_PAD, D), jnp.float32),
    )(x, do0, do1, w)


def _mid_pre(p0, p1, do0, do1, di0, di1, b, w, dout):
    return pl.pallas_call(
        _mid_pre_body,
        grid=(_GRID,),
        in_specs=[_row_spec(D), _row_spec(D), _row_spec(16), _row_spec(16),
                  _row_spec(16), _row_spec(16), _full_spec(1, D),
                  _full_spec(D, dout)],
        out_specs=_row_spec(dout),
        out_shape=jax.ShapeDtypeStruct((N_PAD, dout), jnp.float32),
    )(p0, p1, do0, do1, di0, di1, b, w)


def _final(p0, p1, di0, di1, b):
    return pl.pallas_call(
        _final_body,
        grid=(_GRID,),
        in_specs=[_row_spec(D2), _row_spec(D2), _row_spec(16), _row_spec(16),
                  _full_spec(1, D2)],
        out_specs=_row_spec(D2),
        out_shape=jax.ShapeDtypeStruct((N_PAD, D2), jnp.float32),
    )(p0, p1, di0, di1, b)


def kernel(features, edge_index, W0, b0, W1, b1, W2, b2):
    pad = jnp.full((E_PAD - E,), N, dtype=jnp.int32)
    src = jnp.concatenate([edge_index[0], pad]).reshape(E_PAD // CHUNK, CHUNK)
    dst = jnp.concatenate([edge_index[1], pad]).reshape(E_PAD // CHUNK, CHUNK)

    x = jnp.pad(features, ((0, N_PAD - N), (0, 0)))
    w2 = jnp.pad(W2, ((0, 0), (0, D2 - NCLS)))
    b0r = b0.reshape(1, D)
    b1r = b1.reshape(1, D)
    b2r = jnp.pad(b2, (0, D2 - NCLS)).reshape(1, D2)

    ones16 = jnp.ones((CHUNK, 16), jnp.float32)
    z16 = jnp.zeros((RPT, 16), jnp.float32)
    z128 = jnp.zeros((RPT, D), jnp.float32)
    z64 = jnp.zeros((RPT, D2), jnp.float32)

    deg = _degrees(src, dst, ones16, z16)
    do0, do1 = deg[0, 0], deg[1, 0]
    di0, di1 = deg[0, 1], deg[1, 1]

    y0 = _first_pre(x, do0, do1, W0)
    p = _agg128(y0, src, dst, z128)
    y1 = _mid_pre(p[0], p[1], do0, do1, di0, di1, b0r, W1, D)
    p = _agg128(y1, src, dst, z128)
    y2 = _mid_pre(p[0], p[1], do0, do1, di0, di1, b1r, w2, D2)
    p = _agg64(y2, src, dst, z64)
    out = _final(p[0], p[1], di0, di1, b2r)
    return out[:N, :NCLS]
